# baseline (speedup 1.0000x reference)
import jax
import jax.numpy as jnp
from jax.experimental import pallas as pl
from jax.experimental.pallas import tpu as pltpu


def _gcn_body(adj_ref, nodes_ref, w_ref, out_ref):
    a = adj_ref[...].astype(jnp.bfloat16)
    x = nodes_ref[...].astype(jnp.bfloat16)
    agg = jax.lax.dot_general(
        a, x, (((2,), (1,)), ((0,), (0,))),
        preferred_element_type=jnp.float32).astype(jnp.bfloat16)
    w = w_ref[...].astype(jnp.bfloat16)
    out_ref[...] = jax.lax.dot_general(
        agg, w, (((2,), (1,)), ((0,), (0,))),
        preferred_element_type=jnp.float32)


def kernel(adjacency, nodes, weights):
    adjacency = adjacency.astype(jnp.float32)
    nodes = nodes.astype(jnp.float32)
    weights = weights.astype(jnp.float32)

    B, W, N, _ = adjacency.shape
    Fin = nodes.shape[-1]
    Wp, _, Fout = weights.shape
    w_used = weights[Wp - W:, :, :]

    H = 2
    G = W // H
    adj_f = adjacency.reshape(B, H, G, N, N)
    nodes_f = nodes.reshape(B, H, G, N, Fin)
    w_f = w_used.reshape(H, G, Fin, Fout)

    out = pl.pallas_call(
        _gcn_body,
        grid=(H, B),
        in_specs=[
            pl.BlockSpec((None, None, G, N, N), lambda h, b: (b, h, 0, 0, 0)),
            pl.BlockSpec((None, None, G, N, Fin), lambda h, b: (b, h, 0, 0, 0)),
            pl.BlockSpec((None, G, Fin, Fout), lambda h, b: (h, 0, 0, 0)),
        ],
        out_specs=pl.BlockSpec((None, None, G, N, Fout),
                               lambda h, b: (b, h, 0, 0, 0)),
        out_shape=jax.ShapeDtypeStruct((B, H, G, N, Fout), jnp.float32),
        compiler_params=pltpu.CompilerParams(
            dimension_semantics=("parallel", "parallel")),
    )(adj_f, nodes_f, w_f)
    return out.reshape(B, W, N, Fout)


# speedup vs baseline: 1.1332x; 1.1332x over previous
import jax
import jax.numpy as jnp
from jax.experimental import pallas as pl
from jax.experimental.pallas import tpu as pltpu


def _compute(adj_buf, nodes_buf, w_ref, out_buf, i):
    a = adj_buf[i].astype(jnp.bfloat16)
    x = nodes_buf[i].astype(jnp.bfloat16)
    agg = jax.lax.dot_general(
        a, x, (((2,), (1,)), ((0,), (0,))),
        preferred_element_type=jnp.float32).astype(jnp.bfloat16)
    w = w_ref[...].astype(jnp.bfloat16)
    out_buf[i] = jax.lax.dot_general(
        agg, w, (((2,), (1,)), ((0,), (0,))),
        preferred_element_type=jnp.float32)


def _make_body(bpc):
    def _body(adj_hbm, nodes_hbm, w_ref, out_hbm,
              adj_buf, nodes_buf, out_buf, adj_sem, nodes_sem, out_sem):
        c = pl.program_id(0)
        for i in range(bpc):
            pltpu.make_async_copy(adj_hbm.at[c, i], adj_buf.at[i],
                                  adj_sem.at[i]).start()
            pltpu.make_async_copy(nodes_hbm.at[c, i], nodes_buf.at[i],
                                  nodes_sem.at[i]).start()
        for i in range(bpc):
            pltpu.make_async_copy(adj_hbm.at[c, i], adj_buf.at[i],
                                  adj_sem.at[i]).wait()
            pltpu.make_async_copy(nodes_hbm.at[c, i], nodes_buf.at[i],
                                  nodes_sem.at[i]).wait()
            _compute(adj_buf, nodes_buf, w_ref, out_buf, i)
            pltpu.make_async_copy(out_buf.at[i], out_hbm.at[c, i],
                                  out_sem.at[i]).start()
        for i in range(bpc):
            pltpu.make_async_copy(out_buf.at[i], out_hbm.at[c, i],
                                  out_sem.at[i]).wait()
    return _body


def kernel(adjacency, nodes, weights):
    adjacency = adjacency.astype(jnp.float32)
    nodes = nodes.astype(jnp.float32)
    weights = weights.astype(jnp.float32)

    B, W, N, _ = adjacency.shape
    Fin = nodes.shape[-1]
    Wp, _, Fout = weights.shape
    w_used = weights[Wp - W:, :, :]

    ncores = 2 if B % 2 == 0 else 1
    bpc = B // ncores
    adj_r = adjacency.reshape(ncores, bpc, W, N, N)
    nodes_r = nodes.reshape(ncores, bpc, W, N, Fin)

    out = pl.pallas_call(
        _make_body(bpc),
        grid=(ncores,),
        in_specs=[
            pl.BlockSpec(memory_space=pl.ANY),
            pl.BlockSpec(memory_space=pl.ANY),
            pl.BlockSpec((W, Fin, Fout), lambda c: (0, 0, 0)),
        ],
        out_specs=pl.BlockSpec(memory_space=pl.ANY),
        out_shape=jax.ShapeDtypeStruct((ncores, bpc, W, N, Fout), jnp.float32),
        scratch_shapes=[
            pltpu.VMEM((bpc, W, N, N), jnp.float32),
            pltpu.VMEM((bpc, W, N, Fin), jnp.float32),
            pltpu.VMEM((bpc, W, N, Fout), jnp.float32),
            pltpu.SemaphoreType.DMA((bpc,)),
            pltpu.SemaphoreType.DMA((bpc,)),
            pltpu.SemaphoreType.DMA((bpc,)),
        ],
        compiler_params=pltpu.CompilerParams(
            dimension_semantics=("parallel",)),
    )(adj_r, nodes_r, w_used)
    return out.reshape(B, W, N, Fout)


# speedup vs baseline: 1.1811x; 1.0423x over previous
import jax
import jax.numpy as jnp
from jax.experimental import pallas as pl
from jax.experimental.pallas import tpu as pltpu


def _make_body(spc):
    def _body(adj_ref, nodes_ref, w_ref, out_hbm, out_buf, out_sem):
        b = pl.program_id(0)
        slot = jax.lax.rem(b, spc)
        a = adj_ref[...].astype(jnp.bfloat16)
        x = nodes_ref[...].astype(jnp.bfloat16)
        agg = jax.lax.dot_general(
            a, x, (((2,), (1,)), ((0,), (0,))),
            preferred_element_type=jnp.float32).astype(jnp.bfloat16)
        w = w_ref[...].astype(jnp.bfloat16)
        out_buf[slot] = jax.lax.dot_general(
            agg, w, (((2,), (1,)), ((0,), (0,))),
            preferred_element_type=jnp.float32)
        pltpu.make_async_copy(out_buf.at[slot], out_hbm.at[b],
                              out_sem.at[slot]).start()

        @pl.when(slot == spc - 1)
        def _():
            for s in range(spc):
                pltpu.make_async_copy(out_buf.at[s], out_hbm.at[b],
                                      out_sem.at[s]).wait()
    return _body


def kernel(adjacency, nodes, weights):
    adjacency = adjacency.astype(jnp.float32)
    nodes = nodes.astype(jnp.float32)
    weights = weights.astype(jnp.float32)

    B, W, N, _ = adjacency.shape
    Fin = nodes.shape[-1]
    Wp, _, Fout = weights.shape
    w_used = weights[Wp - W:, :, :]

    ncores = 2 if B % 2 == 0 else 1
    spc = B // ncores

    return pl.pallas_call(
        _make_body(spc),
        grid=(B,),
        in_specs=[
            pl.BlockSpec((None, W, N, N), lambda b: (b, 0, 0, 0)),
            pl.BlockSpec((None, W, N, Fin), lambda b: (b, 0, 0, 0)),
            pl.BlockSpec((W, Fin, Fout), lambda b: (0, 0, 0)),
        ],
        out_specs=pl.BlockSpec(memory_space=pl.ANY),
        out_shape=jax.ShapeDtypeStruct((B, W, N, Fout), jnp.float32),
        scratch_shapes=[
            pltpu.VMEM((spc, W, N, Fout), jnp.float32),
            pltpu.SemaphoreType.DMA((spc,)),
        ],
        compiler_params=pltpu.CompilerParams(
            dimension_semantics=("parallel",)),
    )(adjacency, nodes, w_used)
